# baseline (speedup 1.0000x reference)
import functools

import jax
import jax.numpy as jnp
from jax import lax
from jax.experimental import pallas as pl
from jax.experimental.pallas import tpu as pltpu
from jax.experimental.pallas import tpu_sc as plsc

_K = 5
_C = 3
_EPS = 1e-06
_ALPHA = 0.5
_NC, _NS, _L = 2, 16, 16
_NW = _NC * _NS
_CH = 8192


def _sc_body(tab_hbm, asg_hbm, img_hbm, out_hbm, tab,
             asg_a, a0, a1, a2, asg_b, b0, b1, b2,
             sem_in_a, sem_in_b, sem_out_a, sem_out_b, *, hw):
    pltpu.sync_copy(tab_hbm, tab)

    sets = ((asg_a, (a0, a1, a2), sem_in_a, sem_out_a),
            (asg_b, (b0, b1, b2), sem_in_b, sem_out_b))
    wid = lax.axis_index("s") * _NC + lax.axis_index("c")
    b = wid // 2
    base = (wid % 2) * (hw // 2)
    nch = hw // 2 // _CH

    out_pending = [0, 0]

    def start_in(t):
        asg_v, bufs, sem_in, sem_out = sets[t % 2]
        if out_pending[t % 2]:
            for c in range(_C):
                pltpu.make_async_copy(
                    bufs[c],
                    out_hbm.at[pl.ds((b * _C + c) * hw, _CH)],
                    sem_out).wait()
            out_pending[t % 2] = 0
        off = base + t * _CH
        pltpu.async_copy(asg_hbm.at[pl.ds(b * hw + off, _CH)], asg_v, sem_in)
        for c in range(_C):
            pltpu.async_copy(img_hbm.at[pl.ds((b * _C + c) * hw + off, _CH)],
                             bufs[c], sem_in)

    start_in(0)
    for t in range(nch):
        asg_v, bufs, sem_in, sem_out = sets[t % 2]
        if t + 1 < nch:
            start_in(t + 1)
        pltpu.make_async_copy(asg_hbm.at[pl.ds(base, _CH)], asg_v,
                              sem_in).wait()
        for c in range(_C):
            pltpu.make_async_copy(img_hbm.at[pl.ds(base, _CH)], bufs[c],
                                  sem_in).wait()

        @plsc.parallel_loop(0, _CH, _L, unroll=4)
        def _chunk(j):
            a = asg_v[pl.ds(j, _L)]
            for c in range(_C):
                x = bufs[c][pl.ds(j, _L)]
                sc = plsc.load_gather(tab, [a + (c * 8)])
                of = plsc.load_gather(tab, [a + (c * 8 + 24)])
                y = jnp.minimum(jnp.maximum(x * sc + of, 0.0), 1.0)
                bufs[c][pl.ds(j, _L)] = y

        off = base + t * _CH
        for c in range(_C):
            pltpu.async_copy(bufs[c],
                             out_hbm.at[pl.ds((b * _C + c) * hw + off, _CH)],
                             sem_out)
        out_pending[t % 2] = 1

    for s in range(2):
        if out_pending[s]:
            asg_v, bufs, sem_in, sem_out = sets[s]
            for c in range(_C):
                pltpu.make_async_copy(
                    bufs[c],
                    out_hbm.at[pl.ds((b * _C + c) * hw, _CH)],
                    sem_out).wait()


@jax.jit
def _run_sc(img, asg, mu_s, sig_s, mu_t, sig_t):
    B, C, H, W = img.shape
    hw = H * W
    r = sig_t / (sig_s + _EPS)
    scale = _ALPHA * r + (1.0 - _ALPHA)
    offset = _ALPHA * (mu_t - mu_s * r)
    tab8 = jnp.zeros((2, _C, 8), jnp.float32)
    tab8 = tab8.at[0, :, :_K].set(scale.T).at[1, :, :_K].set(offset.T)
    tab_flat = tab8.reshape(48)
    mesh = plsc.VectorSubcoreMesh(core_axis_name="c", subcore_axis_name="s")
    out = pl.kernel(
        functools.partial(_sc_body, hw=hw),
        out_type=jax.ShapeDtypeStruct((B * C * hw,), jnp.float32),
        mesh=mesh,
        compiler_params=pltpu.CompilerParams(needs_layout_passes=False),
        scratch_types=[
            pltpu.VMEM((48,), jnp.float32),
            pltpu.VMEM((_CH,), jnp.int32),
            pltpu.VMEM((_CH,), jnp.float32),
            pltpu.VMEM((_CH,), jnp.float32),
            pltpu.VMEM((_CH,), jnp.float32),
            pltpu.VMEM((_CH,), jnp.int32),
            pltpu.VMEM((_CH,), jnp.float32),
            pltpu.VMEM((_CH,), jnp.float32),
            pltpu.VMEM((_CH,), jnp.float32),
            pltpu.SemaphoreType.DMA,
            pltpu.SemaphoreType.DMA,
            pltpu.SemaphoreType.DMA,
            pltpu.SemaphoreType.DMA,
        ],
    )(tab_flat, asg.reshape(B * hw), img.reshape(B * C * hw))
    return out.reshape(B, C, H, W)


def kernel(source_images, source_assignments, source_stats_means,
           source_stats_stds, target_stats_means, target_stats_stds):
    asg = source_assignments.astype(jnp.int32)
    return _run_sc(source_images, asg, source_stats_means, source_stats_stds,
                   target_stats_means, target_stats_stds)


# speedup vs baseline: 4.3909x; 4.3909x over previous
import functools

import jax
import jax.numpy as jnp
from jax.experimental import pallas as pl
from jax.experimental.pallas import tpu as pltpu

_K = 5
_C = 3
_EPS = 1e-06
_ALPHA = 0.5


def _body(mu_s_ref, sig_s_ref, mu_t_ref, sig_t_ref, asg_ref, img_ref, out_ref,
          *, bb, h):
    scale = [[None] * _C for _ in range(_K)]
    offset = [[None] * _C for _ in range(_K)]
    for k in range(_K):
        for c in range(_C):
            r = sig_t_ref[k, c] / (sig_s_ref[k, c] + _EPS)
            scale[k][c] = _ALPHA * r + (1.0 - _ALPHA)
            offset[k][c] = _ALPHA * (mu_t_ref[k, c] - mu_s_ref[k, c] * r)

    for b in range(bb):
        for s in range(h // 8):
            rows = pl.ds(s * 8, 8)
            asg = asg_ref[b, rows, :]
            masks = [asg == k for k in range(_K - 1)]
            for c in range(_C):
                sc = jnp.full(asg.shape, scale[_K - 1][c], jnp.float32)
                of = jnp.full(asg.shape, offset[_K - 1][c], jnp.float32)
                for k in range(_K - 2, -1, -1):
                    sc = jnp.where(masks[k], scale[k][c], sc)
                    of = jnp.where(masks[k], offset[k][c], of)
                x = img_ref[b, c, rows, :]
                out_ref[b, c, rows, :] = jnp.clip(x * sc + of, 0.0, 1.0)


@functools.partial(jax.jit, static_argnames=("bb",))
def _run(img, asg, mu_s, sig_s, mu_t, sig_t, bb=2):
    B, C, H, W = img.shape
    grid = (B // bb,)
    stats_spec = pl.BlockSpec(memory_space=pltpu.SMEM)
    return pl.pallas_call(
        functools.partial(_body, bb=bb, h=H),
        grid=grid,
        in_specs=[
            stats_spec,
            stats_spec,
            stats_spec,
            stats_spec,
            pl.BlockSpec((bb, H, W), lambda b: (b, 0, 0)),
            pl.BlockSpec((bb, C, H, W), lambda b: (b, 0, 0, 0)),
        ],
        out_specs=pl.BlockSpec((bb, C, H, W), lambda b: (b, 0, 0, 0)),
        out_shape=jax.ShapeDtypeStruct((B, C, H, W), jnp.float32),
        compiler_params=pltpu.CompilerParams(
            dimension_semantics=("parallel",),
        ),
    )(mu_s, sig_s, mu_t, sig_t, asg, img)


def kernel(source_images, source_assignments, source_stats_means,
           source_stats_stds, target_stats_means, target_stats_stds):
    asg = source_assignments.astype(jnp.int32)
    return _run(source_images, asg, source_stats_means, source_stats_stds,
                target_stats_means, target_stats_stds)
